# baseline (speedup 1.0000x reference)
import functools

import jax
import jax.numpy as jnp
from jax import lax
from jax.experimental import pallas as pl
from jax.experimental.pallas import tpu as pltpu
from jax.experimental.pallas import tpu_sc as plsc

N = 50000
D = 4
K = 4
E = 50000
EPP = 12500
NW = 32
CE = 1568
NSTEP = 49
PADN = NW * CE
BLK = 1024
F1 = 64
F2 = 128
NBUF = 6

_ILV = plsc.PackFormat.INTERLEAVED
_BF = jnp.bfloat16


def _tca_body(x_ref, wp_ref, bp_ref, w2a_ref, b2a_ref, w2b_ref, b2b_ref,
              o1_ref, o2_ref):
    x = x_ref[...]
    o1_ref[...] = jax.lax.dot_general(
        x, wp_ref[...], (((1,), (0,)), ((), ())),
        preferred_element_type=jnp.float32) + bp_ref[...]
    h = jnp.maximum(jax.lax.dot_general(
        x, w2a_ref[...], (((1,), (0,)), ((), ())),
        preferred_element_type=jnp.float32) + b2a_ref[...], 0.0)
    y = jax.lax.dot_general(
        h, w2b_ref[...], (((1,), (0,)), ((), ())),
        preferred_element_type=jnp.float32) + b2b_ref[...]
    o2_ref[...] = y.astype(_BF)


def _tc_a(emb, wpT, bp1, w2aT, b2a1, w2bT, b2b1):
    g = PADN // BLK
    return pl.pallas_call(
        _tca_body,
        grid=(g,),
        in_specs=[
            pl.BlockSpec((BLK, F2), lambda i: (i, 0)),
            pl.BlockSpec((F2, F1), lambda i: (0, 0)),
            pl.BlockSpec((1, F1), lambda i: (0, 0)),
            pl.BlockSpec((F2, F2), lambda i: (0, 0)),
            pl.BlockSpec((1, F2), lambda i: (0, 0)),
            pl.BlockSpec((F2, F2), lambda i: (0, 0)),
            pl.BlockSpec((1, F2), lambda i: (0, 0)),
        ],
        out_specs=[
            pl.BlockSpec((BLK, F1), lambda i: (i, 0)),
            pl.BlockSpec((BLK, F2), lambda i: (i, 0)),
        ],
        out_shape=[
            jax.ShapeDtypeStruct((PADN, F1), jnp.float32),
            jax.ShapeDtypeStruct((PADN, F2), _BF),
        ],
    )(emb, wpT, bp1, w2aT, b2a1, w2bT, b2b1)


_MESH = plsc.VectorSubcoreMesh(core_axis_name="c", subcore_axis_name="s")
_SC_PARAMS = pltpu.CompilerParams(use_tc_tiling_on_sc=False,
                                  needs_layout_passes=False)


def _wid():
    return lax.axis_index("s") * 2 + lax.axis_index("c")


def _run_pipeline(nbuf, start_gather, wait_gather, compute, start_store,
                  wait_store):
    def body(i, c):
        for b in range(nbuf):
            start_gather(nbuf * i + b, b)
        for b in range(nbuf):
            wait_gather(b)
            compute(nbuf * i + b, b)
            start_store(nbuf * i + b, b)
        for b in range(nbuf):
            wait_store(b)
        return c

    lax.fori_loop(0, (NSTEP - 1) // nbuf, body, 0)
    j = NSTEP - 1
    start_gather(j, 0)
    wait_gather(0)
    compute(j, 0)
    start_store(j, 0)
    wait_store(0)


NBUF_F = 4


@functools.partial(
    pl.kernel, mesh=_MESH, compiler_params=_SC_PARAMS,
    out_type=[
        jax.ShapeDtypeStruct((D * PADN, F1), jnp.float32),
        jax.ShapeDtypeStruct((PADN, F2), _BF),
    ],
    scratch_types=[
        pltpu.VMEM((NSTEP, 128), jnp.int32),
        pltpu.VMEM((NSTEP, 128), jnp.int32),
        [pltpu.VMEM((128, F1), jnp.float32)] * NBUF_F,
        [pltpu.VMEM((128, F1), jnp.float32)] * NBUF_F,
        [pltpu.VMEM((128, F2), _BF)] * NBUF_F,
        [pltpu.VMEM((32, F2), _BF)] * NBUF_F,
        [pltpu.SemaphoreType.DMA] * NBUF_F,
        [pltpu.SemaphoreType.DMA] * NBUF_F,
        [pltpu.SemaphoreType.DMA] * NBUF_F,
        [pltpu.SemaphoreType.DMA] * NBUF_F,
    ])
def _sc_loo_edge(emb_hbm, emb2_hbm, gidx_hbm, tidx_hbm, t_hbm, s_hbm,
                 gv, sv, rows, loo, rows2, sbuf, sg, ss, sg2, ss2):
    w = _wid()
    pltpu.sync_copy(gidx_hbm.at[w], gv)
    pltpu.sync_copy(tidx_hbm.at[w], sv)
    dummy = emb_hbm.at[pl.ds(0, 128)]
    dummy2 = emb2_hbm.at[pl.ds(0, 128)]
    dummy2s = emb2_hbm.at[pl.ds(0, 32)]

    def start_gather(j, b):
        pltpu.async_copy(emb_hbm.at[gv.at[j]], rows[b], sg[b])
        pltpu.async_copy(emb2_hbm.at[gv.at[j]], rows2[b], sg2[b])

    def wait_gather(b):
        pltpu.make_async_copy(dummy, rows[b], sg[b]).wait()
        pltpu.make_async_copy(dummy2, rows2[b], sg2[b]).wait()

    def compute(j, b):
        rp, lp = rows[b], loo[b]
        r2, op = rows2[b], sbuf[b]

        def edge(jj, c2):
            for u in range(2):
                e = 2 * jj + u
                for g in range(F1 // 16):
                    sl = pl.ds(g * 16, 16)
                    a = rp[4 * e, sl]
                    bb = rp[4 * e + 1, sl]
                    cc = rp[4 * e + 2, sl]
                    dd = rp[4 * e + 3, sl]
                    ab = a * bb
                    cd = cc * dd
                    lp[4 * e, sl] = bb * cd
                    lp[4 * e + 1, sl] = a * cd
                    lp[4 * e + 2, sl] = ab * dd
                    lp[4 * e + 3, sl] = ab * cc
                for g in range(F2 // 32):
                    sl = pl.ds(g * 32, 32)
                    a0, a1 = plsc.unpack(r2[4 * e, sl], format=_ILV)
                    b0, b1 = plsc.unpack(r2[4 * e + 1, sl], format=_ILV)
                    c0, c1 = plsc.unpack(r2[4 * e + 2, sl], format=_ILV)
                    d0, d1 = plsc.unpack(r2[4 * e + 3, sl], format=_ILV)
                    s0 = jnp.maximum((a0 + b0) + (c0 + d0), 0.0)
                    s1 = jnp.maximum((a1 + b1) + (c1 + d1), 0.0)
                    op[e, sl] = plsc.pack(s0, s1, format=_ILV)
            return c2

        lax.fori_loop(0, 16, edge, 0)

    def start_store(j, b):
        pltpu.async_copy(loo[b], t_hbm.at[sv.at[j]], ss[b])
        pltpu.async_copy(sbuf[b], s_hbm.at[pl.ds(w * CE + j * 32, 32)],
                         ss2[b])

    def wait_store(b):
        pltpu.make_async_copy(dummy, loo[b], ss[b]).wait()
        pltpu.make_async_copy(dummy2s, sbuf[b], ss2[b]).wait()

    _run_pipeline(NBUF_F, start_gather, wait_gather, compute, start_store,
                  wait_store)


def _make_sum_kernel(with_relu):
    @functools.partial(
        pl.kernel, mesh=_MESH, compiler_params=_SC_PARAMS,
        out_type=jax.ShapeDtypeStruct((PADN, F2), _BF),
        scratch_types=[
            pltpu.VMEM((NSTEP, 128), jnp.int32),
            [pltpu.VMEM((128, F2), _BF)] * NBUF,
            [pltpu.VMEM((32, F2), _BF)] * NBUF,
            [pltpu.SemaphoreType.DMA] * NBUF,
            [pltpu.SemaphoreType.DMA] * NBUF,
        ])
    def sum_kernel(tab_hbm, idx_hbm, out_hbm, gv, rows, ob, sg, ss):
        w = _wid()
        pltpu.sync_copy(idx_hbm.at[w], gv)
        dummy_g = tab_hbm.at[pl.ds(0, 128)]
        dummy_s = tab_hbm.at[pl.ds(0, 32)]

        def start_gather(j, b):
            pltpu.async_copy(tab_hbm.at[gv.at[j]], rows[b], sg[b])

        def wait_gather(b):
            pltpu.make_async_copy(dummy_g, rows[b], sg[b]).wait()

        def compute(j, b):
            rp, op = rows[b], ob[b]

            def row(jj, c2):
                for u in range(4):
                    e = 4 * jj + u
                    for g in range(F2 // 32):
                        sl = pl.ds(g * 32, 32)
                        a0, a1 = plsc.unpack(rp[4 * e, sl], format=_ILV)
                        b0, b1 = plsc.unpack(rp[4 * e + 1, sl], format=_ILV)
                        c0, c1 = plsc.unpack(rp[4 * e + 2, sl], format=_ILV)
                        d0, d1 = plsc.unpack(rp[4 * e + 3, sl], format=_ILV)
                        s0 = (a0 + b0) + (c0 + d0)
                        s1 = (a1 + b1) + (c1 + d1)
                        if with_relu:
                            s0 = jnp.maximum(s0, 0.0)
                            s1 = jnp.maximum(s1, 0.0)
                        op[e, sl] = plsc.pack(s0, s1, format=_ILV)
                return c2

            lax.fori_loop(0, 8, row, 0)

        def start_store(j, b):
            pltpu.async_copy(ob[b], out_hbm.at[pl.ds(w * CE + j * 32, 32)],
                             ss[b])

        def wait_store(b):
            pltpu.make_async_copy(dummy_s, ob[b], ss[b]).wait()

        _run_pipeline(NBUF, start_gather, wait_gather, compute, start_store,
                      wait_store)

    return sum_kernel


_sc_nodesum = _make_sum_kernel(False)


def _tcb_body(t_ref, r_ref, wq_ref, bq_ref, o_ref):
    acc = jnp.tanh((2.0 / 3.0) * t_ref[0])
    for d in range(1, D):
        acc = acc + jnp.tanh((2.0 / 3.0) * t_ref[d])
    q = jax.lax.dot_general(acc, wq_ref[...], (((1,), (0,)), ((), ())),
                            preferred_element_type=jnp.float32)
    o_ref[...] = jnp.maximum(
        0.25 * q + bq_ref[...] + 0.25 * r_ref[...].astype(jnp.float32), 0.0)


def _tc_b(t, r, wqT, bq1):
    g = (N + BLK - 1) // BLK
    return pl.pallas_call(
        _tcb_body,
        grid=(g,),
        in_specs=[
            pl.BlockSpec((D, BLK, F1), lambda i: (0, i, 0)),
            pl.BlockSpec((BLK, F2), lambda i: (i, 0)),
            pl.BlockSpec((F1, F2), lambda i: (0, 0)),
            pl.BlockSpec((1, F2), lambda i: (0, 0)),
        ],
        out_specs=pl.BlockSpec((BLK, F2), lambda i: (i, 0)),
        out_shape=jax.ShapeDtypeStruct((N, F2), jnp.float32),
    )(t, r, wqT, bq1)


def kernel(embedding, Wp, bp, Wp2a, bp2a, Wp2b, bp2b, Wq, bq,
           edge_nodes, node_edges):
    f32 = jnp.float32
    i32 = jnp.int32

    wpT = jnp.zeros((F2, F1), f32).at[:, :50].set(Wp.astype(f32).T)
    bp1 = jnp.zeros((1, F1), f32).at[0, :50].set(bp.astype(f32))
    w2aT = Wp2a.astype(f32).T
    b2a1 = bp2a.astype(f32).reshape(1, F2)
    w2bT = Wp2b.astype(f32).T
    b2b1 = bp2b.astype(f32).reshape(1, F2)
    wqT = jnp.zeros((F1, F2), f32).at[:50, :].set(Wq.astype(f32).T)
    bq1 = bq.astype(f32).reshape(1, F2)

    en = edge_nodes.astype(i32)
    en_pad = jnp.full((PADN, K), N, i32).at[:E].set(en)
    gidx = en_pad.reshape(NW, NSTEP, 128)
    d_of_e = jnp.minimum(jnp.arange(PADN, dtype=i32) // EPP, D - 1)
    tidx = (d_of_e[:, None] * PADN + en_pad).reshape(NW, NSTEP, 128)
    ne = node_edges.astype(i32)
    ne_pad = jnp.zeros((PADN, K), i32).at[:N].set(ne)
    nidx = ne_pad.reshape(NW, NSTEP, 128)

    emb_new, emb_new2 = _tc_a(embedding.astype(f32), wpT, bp1, w2aT, b2a1,
                              w2bT, b2b1)
    t_flat, s = _sc_loo_edge(emb_new, emb_new2, gidx, tidx)
    r = _sc_nodesum(s, nidx)
    t = t_flat.reshape(D, PADN, F1)
    return _tc_b(t, r, wqT, bq1)


# speedup vs baseline: 1.0369x; 1.0369x over previous
import functools

import jax
import jax.numpy as jnp
from jax import lax
from jax.experimental import pallas as pl
from jax.experimental.pallas import tpu as pltpu
from jax.experimental.pallas import tpu_sc as plsc

N = 50000
D = 4
K = 4
E = 50000
EPP = 12500
NW = 32
CE = 1568
NSTEP = 49
PADN = NW * CE
BLK = 1024
F1 = 64
F2 = 128
NBUF = 6

_ILV = plsc.PackFormat.INTERLEAVED
_BF = jnp.bfloat16


def _tca_body(x_ref, wp_ref, bp_ref, w2a_ref, b2a_ref, w2b_ref, b2b_ref,
              o1_ref, o2_ref):
    x = x_ref[...]
    o1_ref[...] = jax.lax.dot_general(
        x, wp_ref[...], (((1,), (0,)), ((), ())),
        preferred_element_type=jnp.float32) + bp_ref[...]
    h = jnp.maximum(jax.lax.dot_general(
        x, w2a_ref[...], (((1,), (0,)), ((), ())),
        preferred_element_type=jnp.float32) + b2a_ref[...], 0.0)
    y = jax.lax.dot_general(
        h, w2b_ref[...], (((1,), (0,)), ((), ())),
        preferred_element_type=jnp.float32) + b2b_ref[...]
    o2_ref[...] = y.astype(_BF)


def _tc_a(emb, wpT, bp1, w2aT, b2a1, w2bT, b2b1):
    g = PADN // BLK
    return pl.pallas_call(
        _tca_body,
        grid=(g,),
        in_specs=[
            pl.BlockSpec((BLK, F2), lambda i: (i, 0)),
            pl.BlockSpec((F2, F1), lambda i: (0, 0)),
            pl.BlockSpec((1, F1), lambda i: (0, 0)),
            pl.BlockSpec((F2, F2), lambda i: (0, 0)),
            pl.BlockSpec((1, F2), lambda i: (0, 0)),
            pl.BlockSpec((F2, F2), lambda i: (0, 0)),
            pl.BlockSpec((1, F2), lambda i: (0, 0)),
        ],
        out_specs=[
            pl.BlockSpec((BLK, F1), lambda i: (i, 0)),
            pl.BlockSpec((BLK, F2), lambda i: (i, 0)),
        ],
        out_shape=[
            jax.ShapeDtypeStruct((PADN, F1), jnp.float32),
            jax.ShapeDtypeStruct((PADN, F2), _BF),
        ],
    )(emb, wpT, bp1, w2aT, b2a1, w2bT, b2b1)


_MESH = plsc.VectorSubcoreMesh(core_axis_name="c", subcore_axis_name="s")
_SC_PARAMS = pltpu.CompilerParams(use_tc_tiling_on_sc=False,
                                  needs_layout_passes=False)


def _wid():
    return lax.axis_index("s") * 2 + lax.axis_index("c")


def _run_pipeline(start_gather, wait_gather, compute, start_store, wait_store):
    def body(i, c):
        for b in range(NBUF):
            start_gather(NBUF * i + b, b)
        for b in range(NBUF):
            wait_gather(b)
            compute(NBUF * i + b, b)
            start_store(NBUF * i + b, b)
        for b in range(NBUF):
            wait_store(b)
        return c

    lax.fori_loop(0, (NSTEP - 1) // NBUF, body, 0)
    j = NSTEP - 1
    start_gather(j, 0)
    wait_gather(0)
    compute(j, 0)
    start_store(j, 0)
    wait_store(0)


@functools.partial(
    pl.kernel, mesh=_MESH, compiler_params=_SC_PARAMS,
    out_type=jax.ShapeDtypeStruct((D * PADN, F1), jnp.float32),
    scratch_types=[
        pltpu.VMEM((NSTEP, 128), jnp.int32),
        pltpu.VMEM((NSTEP, 128), jnp.int32),
        [pltpu.VMEM((128, F1), jnp.float32)] * NBUF,
        [pltpu.VMEM((128, F1), jnp.float32)] * NBUF,
        [pltpu.SemaphoreType.DMA] * NBUF,
        [pltpu.SemaphoreType.DMA] * NBUF,
    ])
def _sc_loo(emb_hbm, gidx_hbm, tidx_hbm, t_hbm, gv, sv, rows, loo, sg, ss):
    w = _wid()
    pltpu.sync_copy(gidx_hbm.at[w], gv)
    pltpu.sync_copy(tidx_hbm.at[w], sv)
    dummy = emb_hbm.at[pl.ds(0, 128)]

    def start_gather(j, b):
        pltpu.async_copy(emb_hbm.at[gv.at[j]], rows[b], sg[b])

    def wait_gather(b):
        pltpu.make_async_copy(dummy, rows[b], sg[b]).wait()

    def compute(j, b):
        rp, lp = rows[b], loo[b]

        def edge(jj, c2):
            for u in range(2):
                e = 2 * jj + u
                for g in range(F1 // 16):
                    sl = pl.ds(g * 16, 16)
                    a = rp[4 * e, sl]
                    bb = rp[4 * e + 1, sl]
                    cc = rp[4 * e + 2, sl]
                    dd = rp[4 * e + 3, sl]
                    ab = a * bb
                    cd = cc * dd
                    lp[4 * e, sl] = bb * cd
                    lp[4 * e + 1, sl] = a * cd
                    lp[4 * e + 2, sl] = ab * dd
                    lp[4 * e + 3, sl] = ab * cc
            return c2

        lax.fori_loop(0, 16, edge, 0)

    def start_store(j, b):
        pltpu.async_copy(loo[b], t_hbm.at[sv.at[j]], ss[b])

    def wait_store(b):
        pltpu.make_async_copy(dummy, loo[b], ss[b]).wait()

    _run_pipeline(start_gather, wait_gather, compute, start_store, wait_store)


def _make_sum_kernel(with_relu):
    @functools.partial(
        pl.kernel, mesh=_MESH, compiler_params=_SC_PARAMS,
        out_type=jax.ShapeDtypeStruct((PADN, F2), _BF),
        scratch_types=[
            pltpu.VMEM((NSTEP, 128), jnp.int32),
            [pltpu.VMEM((128, F2), _BF)] * NBUF,
            [pltpu.VMEM((32, F2), _BF)] * NBUF,
            [pltpu.SemaphoreType.DMA] * NBUF,
            [pltpu.SemaphoreType.DMA] * NBUF,
        ])
    def sum_kernel(tab_hbm, idx_hbm, out_hbm, gv, rows, ob, sg, ss):
        w = _wid()
        pltpu.sync_copy(idx_hbm.at[w], gv)
        dummy_g = tab_hbm.at[pl.ds(0, 128)]
        dummy_s = tab_hbm.at[pl.ds(0, 32)]

        def start_gather(j, b):
            pltpu.async_copy(tab_hbm.at[gv.at[j]], rows[b], sg[b])

        def wait_gather(b):
            pltpu.make_async_copy(dummy_g, rows[b], sg[b]).wait()

        def compute(j, b):
            rp, op = rows[b], ob[b]

            def row(jj, c2):
                for u in range(4):
                    e = 4 * jj + u
                    for g in range(F2 // 32):
                        sl = pl.ds(g * 32, 32)
                        s = ((rp[4 * e, sl] + rp[4 * e + 1, sl])
                             + (rp[4 * e + 2, sl] + rp[4 * e + 3, sl]))
                        if with_relu:
                            s = jnp.maximum(s, jnp.zeros((32,), _BF))
                        op[e, sl] = s
                return c2

            lax.fori_loop(0, 8, row, 0)

        def start_store(j, b):
            pltpu.async_copy(ob[b], out_hbm.at[pl.ds(w * CE + j * 32, 32)],
                             ss[b])

        def wait_store(b):
            pltpu.make_async_copy(dummy_s, ob[b], ss[b]).wait()

        _run_pipeline(start_gather, wait_gather, compute, start_store,
                      wait_store)

    return sum_kernel


_sc_edgesum = _make_sum_kernel(True)
_sc_nodesum = _make_sum_kernel(False)


def _tcb_body(t_ref, r_ref, wq_ref, bq_ref, o_ref):
    acc = jnp.tanh((2.0 / 3.0) * t_ref[0])
    for d in range(1, D):
        acc = acc + jnp.tanh((2.0 / 3.0) * t_ref[d])
    q = jax.lax.dot_general(acc, wq_ref[...], (((1,), (0,)), ((), ())),
                            preferred_element_type=jnp.float32)
    o_ref[...] = jnp.maximum(
        0.25 * q + bq_ref[...] + 0.25 * r_ref[...].astype(jnp.float32), 0.0)


def _tc_b(t, r, wqT, bq1):
    g = (N + BLK - 1) // BLK
    return pl.pallas_call(
        _tcb_body,
        grid=(g,),
        in_specs=[
            pl.BlockSpec((D, BLK, F1), lambda i: (0, i, 0)),
            pl.BlockSpec((BLK, F2), lambda i: (i, 0)),
            pl.BlockSpec((F1, F2), lambda i: (0, 0)),
            pl.BlockSpec((1, F2), lambda i: (0, 0)),
        ],
        out_specs=pl.BlockSpec((BLK, F2), lambda i: (i, 0)),
        out_shape=jax.ShapeDtypeStruct((N, F2), jnp.float32),
    )(t, r, wqT, bq1)


def kernel(embedding, Wp, bp, Wp2a, bp2a, Wp2b, bp2b, Wq, bq,
           edge_nodes, node_edges):
    f32 = jnp.float32
    i32 = jnp.int32

    wpT = jnp.zeros((F2, F1), f32).at[:, :50].set(Wp.astype(f32).T)
    bp1 = jnp.zeros((1, F1), f32).at[0, :50].set(bp.astype(f32))
    w2aT = Wp2a.astype(f32).T
    b2a1 = bp2a.astype(f32).reshape(1, F2)
    w2bT = Wp2b.astype(f32).T
    b2b1 = bp2b.astype(f32).reshape(1, F2)
    wqT = jnp.zeros((F1, F2), f32).at[:50, :].set(Wq.astype(f32).T)
    bq1 = bq.astype(f32).reshape(1, F2)

    en = edge_nodes.astype(i32)
    en_pad = jnp.full((PADN, K), N, i32).at[:E].set(en)
    gidx = en_pad.reshape(NW, NSTEP, 128)
    d_of_e = jnp.minimum(jnp.arange(PADN, dtype=i32) // EPP, D - 1)
    tidx = (d_of_e[:, None] * PADN + en_pad).reshape(NW, NSTEP, 128)
    ne = node_edges.astype(i32)
    ne_pad = jnp.zeros((PADN, K), i32).at[:N].set(ne)
    nidx = ne_pad.reshape(NW, NSTEP, 128)

    emb_new, emb_new2 = _tc_a(embedding.astype(f32), wpT, bp1, w2aT, b2a1,
                              w2bT, b2b1)
    t_flat = _sc_loo(emb_new, gidx, tidx)
    s = _sc_edgesum(emb_new2, gidx)
    r = _sc_nodesum(s, nidx)
    t = t_flat.reshape(D, PADN, F1)
    return _tc_b(t, r, wqT, bq1)


# speedup vs baseline: 1.0384x; 1.0015x over previous
import functools

import jax
import jax.numpy as jnp
from jax import lax
from jax.experimental import pallas as pl
from jax.experimental.pallas import tpu as pltpu
from jax.experimental.pallas import tpu_sc as plsc

N = 50000
D = 4
K = 4
E = 50000
EPP = 12500
NW = 32
CE = 1568
NSTEP = 49
PADN = NW * CE
BLK = 1024
F1 = 64
F2 = 128
NBUF = 6

_BF = jnp.bfloat16


def _tca_body(x_ref, wp_ref, bp_ref, w2a_ref, b2a_ref, w2b_ref, b2b_ref,
              o1_ref, o2_ref):
    x = x_ref[...]
    o1_ref[...] = jax.lax.dot_general(
        x, wp_ref[...], (((1,), (0,)), ((), ())),
        preferred_element_type=jnp.float32) + bp_ref[...]
    h = jnp.maximum(jax.lax.dot_general(
        x, w2a_ref[...], (((1,), (0,)), ((), ())),
        preferred_element_type=jnp.float32) + b2a_ref[...], 0.0)
    y = jax.lax.dot_general(
        h, w2b_ref[...], (((1,), (0,)), ((), ())),
        preferred_element_type=jnp.float32) + b2b_ref[...]
    o2_ref[...] = y.astype(_BF)


def _tc_a(emb, wpT, bp1, w2aT, b2a1, w2bT, b2b1):
    g = PADN // BLK
    return pl.pallas_call(
        _tca_body,
        grid=(g,),
        in_specs=[
            pl.BlockSpec((BLK, F2), lambda i: (i, 0)),
            pl.BlockSpec((F2, F1), lambda i: (0, 0)),
            pl.BlockSpec((1, F1), lambda i: (0, 0)),
            pl.BlockSpec((F2, F2), lambda i: (0, 0)),
            pl.BlockSpec((1, F2), lambda i: (0, 0)),
            pl.BlockSpec((F2, F2), lambda i: (0, 0)),
            pl.BlockSpec((1, F2), lambda i: (0, 0)),
        ],
        out_specs=[
            pl.BlockSpec((BLK, F1), lambda i: (i, 0)),
            pl.BlockSpec((BLK, F2), lambda i: (i, 0)),
        ],
        out_shape=[
            jax.ShapeDtypeStruct((PADN, F1), jnp.float32),
            jax.ShapeDtypeStruct((PADN, F2), _BF),
        ],
    )(emb, wpT, bp1, w2aT, b2a1, w2bT, b2b1)


_MESH = plsc.VectorSubcoreMesh(core_axis_name="c", subcore_axis_name="s")
_SC_PARAMS = pltpu.CompilerParams(use_tc_tiling_on_sc=False,
                                  needs_layout_passes=False)


def _wid():
    return lax.axis_index("s") * 2 + lax.axis_index("c")


def _run_pipeline(start_gather, wait_gather, compute, start_store, wait_store):
    def body(i, c):
        for b in range(NBUF):
            start_gather(NBUF * i + b, b)
        for b in range(NBUF):
            wait_gather(b)
            compute(NBUF * i + b, b)
            start_store(NBUF * i + b, b)
        for b in range(NBUF):
            wait_store(b)
        return c

    lax.fori_loop(0, (NSTEP - 1) // NBUF, body, 0)
    j = NSTEP - 1
    start_gather(j, 0)
    wait_gather(0)
    compute(j, 0)
    start_store(j, 0)
    wait_store(0)


@functools.partial(
    pl.kernel, mesh=_MESH, compiler_params=_SC_PARAMS,
    out_type=jax.ShapeDtypeStruct((D * PADN, F1), jnp.float32),
    scratch_types=[
        pltpu.VMEM((NSTEP, 128), jnp.int32),
        pltpu.VMEM((NSTEP, 128), jnp.int32),
        [pltpu.VMEM((128, F1), jnp.float32)] * NBUF,
        [pltpu.VMEM((128, F1), jnp.float32)] * NBUF,
        [pltpu.SemaphoreType.DMA] * NBUF,
        [pltpu.SemaphoreType.DMA] * NBUF,
    ])
def _sc_loo(emb_hbm, gidx_hbm, tidx_hbm, t_hbm, gv, sv, rows, loo, sg, ss):
    w = _wid()
    pltpu.sync_copy(gidx_hbm.at[w], gv)
    pltpu.sync_copy(tidx_hbm.at[w], sv)
    dummy = emb_hbm.at[pl.ds(0, 128)]

    def start_gather(j, b):
        pltpu.async_copy(emb_hbm.at[gv.at[j]], rows[b], sg[b])

    def wait_gather(b):
        pltpu.make_async_copy(dummy, rows[b], sg[b]).wait()

    def compute(j, b):
        rp, lp = rows[b], loo[b]

        def edge(jj, c2):
            for u in range(2):
                e = 2 * jj + u
                for g in range(F1 // 16):
                    sl = pl.ds(g * 16, 16)
                    a = rp[4 * e, sl]
                    bb = rp[4 * e + 1, sl]
                    cc = rp[4 * e + 2, sl]
                    dd = rp[4 * e + 3, sl]
                    ab = a * bb
                    cd = cc * dd
                    lp[4 * e, sl] = bb * cd
                    lp[4 * e + 1, sl] = a * cd
                    lp[4 * e + 2, sl] = ab * dd
                    lp[4 * e + 3, sl] = ab * cc
            return c2

        lax.fori_loop(0, 16, edge, 0)

    def start_store(j, b):
        pltpu.async_copy(loo[b], t_hbm.at[sv.at[j]], ss[b])

    def wait_store(b):
        pltpu.make_async_copy(dummy, loo[b], ss[b]).wait()

    _run_pipeline(start_gather, wait_gather, compute, start_store, wait_store)


def _make_sum_kernel(with_relu):
    @functools.partial(
        pl.kernel, mesh=_MESH, compiler_params=_SC_PARAMS,
        out_type=jax.ShapeDtypeStruct((PADN, F2), _BF),
        scratch_types=[
            pltpu.VMEM((NSTEP, 128), jnp.int32),
            [pltpu.VMEM((128, F2), _BF)] * NBUF,
            [pltpu.VMEM((32, F2), _BF)] * NBUF,
            [pltpu.SemaphoreType.DMA] * NBUF,
            [pltpu.SemaphoreType.DMA] * NBUF,
        ])
    def sum_kernel(tab_hbm, idx_hbm, out_hbm, gv, rows, ob, sg, ss):
        w = _wid()
        pltpu.sync_copy(idx_hbm.at[w], gv)
        dummy_g = tab_hbm.at[pl.ds(0, 128)]
        dummy_s = tab_hbm.at[pl.ds(0, 32)]

        def start_gather(j, b):
            pltpu.async_copy(tab_hbm.at[gv.at[j]], rows[b], sg[b])

        def wait_gather(b):
            pltpu.make_async_copy(dummy_g, rows[b], sg[b]).wait()

        def compute(j, b):
            rp, op = rows[b], ob[b]

            def row(jj, c2):
                for u in range(4):
                    e = 4 * jj + u
                    for g in range(F2 // 32):
                        sl = pl.ds(g * 32, 32)
                        s = ((rp[4 * e, sl] + rp[4 * e + 1, sl])
                             + (rp[4 * e + 2, sl] + rp[4 * e + 3, sl]))
                        if with_relu:
                            s = jnp.maximum(s, jnp.zeros((32,), _BF))
                        op[e, sl] = s
                return c2

            lax.fori_loop(0, 8, row, 0)

        def start_store(j, b):
            pltpu.async_copy(ob[b], out_hbm.at[pl.ds(w * CE + j * 32, 32)],
                             ss[b])

        def wait_store(b):
            pltpu.make_async_copy(dummy_s, ob[b], ss[b]).wait()

        _run_pipeline(start_gather, wait_gather, compute, start_store,
                      wait_store)

    return sum_kernel


_sc_edgesum = _make_sum_kernel(True)
_sc_nodesum = _make_sum_kernel(False)


def _tcb_body(t_ref, r_ref, wq_ref, bq_ref, o_ref):
    acc = jnp.tanh((2.0 / 3.0) * t_ref[0])
    for d in range(1, D):
        acc = acc + jnp.tanh((2.0 / 3.0) * t_ref[d])
    q = jax.lax.dot_general(acc, wq_ref[...], (((1,), (0,)), ((), ())),
                            preferred_element_type=jnp.float32)
    o_ref[...] = jnp.maximum(
        0.25 * q + bq_ref[...] + 0.25 * r_ref[...].astype(jnp.float32), 0.0)


def _tc_b(t, r, wqT, bq1):
    g = (N + BLK - 1) // BLK
    return pl.pallas_call(
        _tcb_body,
        grid=(g,),
        in_specs=[
            pl.BlockSpec((D, BLK, F1), lambda i: (0, i, 0)),
            pl.BlockSpec((BLK, F2), lambda i: (i, 0)),
            pl.BlockSpec((F1, F2), lambda i: (0, 0)),
            pl.BlockSpec((1, F2), lambda i: (0, 0)),
        ],
        out_specs=pl.BlockSpec((BLK, F2), lambda i: (i, 0)),
        out_shape=jax.ShapeDtypeStruct((N, F2), jnp.float32),
    )(t, r, wqT, bq1)


def kernel(embedding, Wp, bp, Wp2a, bp2a, Wp2b, bp2b, Wq, bq,
           edge_nodes, node_edges):
    f32 = jnp.float32
    i32 = jnp.int32

    wpT = jnp.zeros((F2, F1), f32).at[:, :50].set(Wp.astype(f32).T)
    bp1 = jnp.zeros((1, F1), f32).at[0, :50].set(bp.astype(f32))
    w2aT = Wp2a.astype(f32).T
    b2a1 = bp2a.astype(f32).reshape(1, F2)
    w2bT = Wp2b.astype(f32).T
    b2b1 = bp2b.astype(f32).reshape(1, F2)
    wqT = jnp.zeros((F1, F2), f32).at[:50, :].set(Wq.astype(f32).T)
    bq1 = bq.astype(f32).reshape(1, F2)

    en = edge_nodes.astype(i32)
    en_pad = jnp.full((PADN, K), N, i32).at[:E].set(en)
    gidx = en_pad.reshape(NW, NSTEP, 128)
    d_of_e = jnp.minimum(jnp.arange(PADN, dtype=i32) // EPP, D - 1)
    tidx = (d_of_e[:, None] * PADN + en_pad).reshape(NW, NSTEP, 128)
    ne = node_edges.astype(i32)
    ne_pad = jnp.zeros((PADN, K), i32).at[:N].set(ne)
    nidx = ne_pad.reshape(NW, NSTEP, 128)

    emb_new, emb_new2 = _tc_a(embedding.astype(f32), wpT, bp1, w2aT, b2a1,
                              w2bT, b2b1)
    t_flat = _sc_loo(emb_new, gidx, tidx)
    s = _sc_edgesum(emb_new2, gidx)
    r = _sc_nodesum(s, nidx)
    t = t_flat.reshape(D, PADN, F1)
    return _tc_b(t, r, wqT, bq1)
